# baseline (speedup 1.0000x reference)
import jax
import jax.numpy as jnp
from jax import lax
from jax.experimental import pallas as pl
from jax.experimental.pallas import tpu as pltpu
from jax.experimental.pallas import tpu_sc as plsc

N_ROWS = 1000000
DIM = 64
BATCH = 16384
NW = 32
B_PER_W = BATCH // NW
G = 16
NCH = B_PER_W // G


def _sgns_body(t_ids_hbm, c_ids_hbm, t_w_hbm, c_w_hbm, out_hbm,
               idx_t_v, idx_c_v, q_t_v, q_c_v,
               bt0, bt1, bc0, bc1, part_v, out_v, sem0, sem1):
    wid = lax.axis_index("s") * 2 + lax.axis_index("c")

    pltpu.sync_copy(t_ids_hbm.at[wid], idx_t_v)
    pltpu.sync_copy(c_ids_hbm.at[wid], idx_c_v)
    for i in range(B_PER_W // 16):
        sl = pl.ds(i * 16, 16)
        q_t_v[sl] = lax.shift_right_logical(idx_t_v[sl], 4)
        q_c_v[sl] = lax.shift_right_logical(idx_c_v[sl], 4)

    bufs = ((bt0, bc0, sem0), (bt1, bc1, sem1))

    def fire(j, par):
        bt, bc, sem = bufs[par]
        pltpu.async_copy(t_w_hbm.at[q_t_v.at[pl.ds(j * G, G)]], bt, sem)
        pltpu.async_copy(c_w_hbm.at[q_c_v.at[pl.ds(j * G, G)]], bc, sem)

    def drain(par):
        bt, bc, sem = bufs[par]
        pltpu.make_async_copy(t_w_hbm.at[pl.ds(0, G)], bt, sem).wait()
        pltpu.make_async_copy(c_w_hbm.at[pl.ds(0, G)], bc, sem).wait()

    lane17 = lax.iota(jnp.int32, 16) * 17

    def compute(j, par):
        bt, bc, _ = bufs[par]
        sl = pl.ds(j * G, 16)
        sub_t = lax.bitwise_and(lax.shift_right_logical(idx_t_v[sl], 1), 7)
        sub_c = lax.bitwise_and(lax.shift_right_logical(idx_c_v[sl], 1), 7)
        ofs_t = lax.bitwise_and(idx_t_v[sl], 1) * 64
        ofs_c = lax.bitwise_and(idx_c_v[sl], 1) * 64
        for i in range(16):
            st = sub_t[i]
            sc = sub_c[i]
            ot = ofs_t[i]
            oc = ofs_c[i]
            acc = (bt[i, st, pl.ds(ot, 16)] * bc[i, sc, pl.ds(oc, 16)])
            for k in range(1, 4):
                acc = acc + (bt[i, st, pl.ds(ot + k * 16, 16)]
                             * bc[i, sc, pl.ds(oc + k * 16, 16)])
            plsc.store_scatter(part_v, [lane17 + i], acc)
        s = part_v[pl.ds(0, 16)]
        for l in range(1, 16):
            s = s + part_v[pl.ds(l * 17, 16)]
        out_v[pl.ds(j * G, 16)] = s

    fire(0, 0)
    fire(1, 1)

    def body(g, _):
        j0 = g * 2
        drain(0)
        compute(j0, 0)

        @pl.when(g < NCH // 2 - 1)
        def _():
            fire(j0 + 2, 0)

        drain(1)
        compute(j0 + 1, 1)

        @pl.when(g < NCH // 2 - 1)
        def _():
            fire(j0 + 3, 1)

        return 0

    lax.fori_loop(0, NCH // 2, body, 0)

    pltpu.sync_copy(out_v, out_hbm.at[pl.ds(wid * B_PER_W, B_PER_W)])


@jax.jit
def kernel(t_ids, c_ids, t_in_w, c_out_w):
    t_ids2 = jnp.asarray(t_ids, jnp.int32).reshape(NW, B_PER_W)
    c_ids2 = jnp.asarray(c_ids, jnp.int32).reshape(NW, B_PER_W)
    t_w3 = t_in_w.reshape(N_ROWS // 16, 8, 2 * DIM)
    c_w3 = c_out_w.reshape(N_ROWS // 16, 8, 2 * DIM)

    mesh = plsc.VectorSubcoreMesh(core_axis_name="c", subcore_axis_name="s")
    f = pl.kernel(
        _sgns_body,
        out_type=jax.ShapeDtypeStruct((BATCH,), jnp.float32),
        mesh=mesh,
        compiler_params=pltpu.CompilerParams(needs_layout_passes=False),
        scratch_types=[
            pltpu.VMEM((B_PER_W,), jnp.int32),
            pltpu.VMEM((B_PER_W,), jnp.int32),
            pltpu.VMEM((B_PER_W,), jnp.int32),
            pltpu.VMEM((B_PER_W,), jnp.int32),
            pltpu.VMEM((G, 8, 2 * DIM), jnp.float32),
            pltpu.VMEM((G, 8, 2 * DIM), jnp.float32),
            pltpu.VMEM((G, 8, 2 * DIM), jnp.float32),
            pltpu.VMEM((G, 8, 2 * DIM), jnp.float32),
            pltpu.VMEM((16 * 17,), jnp.float32),
            pltpu.VMEM((B_PER_W,), jnp.float32),
            pltpu.SemaphoreType.DMA,
            pltpu.SemaphoreType.DMA,
        ],
    )
    return f(t_ids2, c_ids2, t_w3, c_w3)


# speedup vs baseline: 1.0194x; 1.0194x over previous
import jax
import jax.numpy as jnp
from jax import lax
from jax.experimental import pallas as pl
from jax.experimental.pallas import tpu as pltpu
from jax.experimental.pallas import tpu_sc as plsc

N_ROWS = 1000000
DIM = 64
BATCH = 16384
NW = 32
B_PER_W = BATCH // NW
G = 16
NCH = B_PER_W // G


def _sgns_body(t_ids_hbm, c_ids_hbm, t_w_hbm, c_w_hbm, out_hbm,
               idx_t_v, idx_c_v, q_t_v, q_c_v,
               bt0, bt1, bc0, bc1, part_v, out_v, sem0, sem1):
    wid = lax.axis_index("s") * 2 + lax.axis_index("c")

    pltpu.sync_copy(t_ids_hbm.at[wid], idx_t_v)
    pltpu.sync_copy(c_ids_hbm.at[wid], idx_c_v)
    for i in range(B_PER_W // 16):
        sl = pl.ds(i * 16, 16)
        q_t_v[sl] = lax.shift_right_logical(idx_t_v[sl], 3)
        q_c_v[sl] = lax.shift_right_logical(idx_c_v[sl], 3)

    bufs = ((bt0, bc0, sem0), (bt1, bc1, sem1))

    def fire(j, par):
        bt, bc, sem = bufs[par]
        pltpu.async_copy(t_w_hbm.at[q_t_v.at[pl.ds(j * G, G)]], bt, sem)
        pltpu.async_copy(c_w_hbm.at[q_c_v.at[pl.ds(j * G, G)]], bc, sem)

    def drain(par):
        bt, bc, sem = bufs[par]
        pltpu.make_async_copy(t_w_hbm.at[pl.ds(0, G)], bt, sem).wait()
        pltpu.make_async_copy(c_w_hbm.at[pl.ds(0, G)], bc, sem).wait()

    lane17 = lax.iota(jnp.int32, 16) * 17

    def compute(j, par):
        bt, bc, _ = bufs[par]
        sl = pl.ds(j * G, 16)
        sub_t = lax.bitwise_and(idx_t_v[sl], 7)
        sub_c = lax.bitwise_and(idx_c_v[sl], 7)
        for i in range(16):
            st = sub_t[i]
            sc = sub_c[i]
            acc = (bt[i, st, pl.ds(0, 16)] * bc[i, sc, pl.ds(0, 16)])
            for k in range(1, 4):
                acc = acc + (bt[i, st, pl.ds(k * 16, 16)]
                             * bc[i, sc, pl.ds(k * 16, 16)])
            plsc.store_scatter(part_v, [lane17 + i], acc)
        s = part_v[pl.ds(0, 16)]
        for l in range(1, 16):
            s = s + part_v[pl.ds(l * 17, 16)]
        out_v[pl.ds(j * G, 16)] = s

    fire(0, 0)
    fire(1, 1)

    def body(g, _):
        j0 = g * 2
        drain(0)
        compute(j0, 0)

        @pl.when(g < NCH // 2 - 1)
        def _():
            fire(j0 + 2, 0)

        drain(1)
        compute(j0 + 1, 1)

        @pl.when(g < NCH // 2 - 1)
        def _():
            fire(j0 + 3, 1)

        return 0

    lax.fori_loop(0, NCH // 2, body, 0)

    pltpu.sync_copy(out_v, out_hbm.at[pl.ds(wid * B_PER_W, B_PER_W)])


@jax.jit
def kernel(t_ids, c_ids, t_in_w, c_out_w):
    t_ids2 = jnp.asarray(t_ids, jnp.int32).reshape(NW, B_PER_W)
    c_ids2 = jnp.asarray(c_ids, jnp.int32).reshape(NW, B_PER_W)
    t_w3 = t_in_w.reshape(N_ROWS // 8, 8, DIM)
    c_w3 = c_out_w.reshape(N_ROWS // 8, 8, DIM)

    mesh = plsc.VectorSubcoreMesh(core_axis_name="c", subcore_axis_name="s")
    f = pl.kernel(
        _sgns_body,
        out_type=jax.ShapeDtypeStruct((BATCH,), jnp.float32),
        mesh=mesh,
        compiler_params=pltpu.CompilerParams(
            needs_layout_passes=False, use_tc_tiling_on_sc=False),
        scratch_types=[
            pltpu.VMEM((B_PER_W,), jnp.int32),
            pltpu.VMEM((B_PER_W,), jnp.int32),
            pltpu.VMEM((B_PER_W,), jnp.int32),
            pltpu.VMEM((B_PER_W,), jnp.int32),
            pltpu.VMEM((G, 8, DIM), jnp.float32),
            pltpu.VMEM((G, 8, DIM), jnp.float32),
            pltpu.VMEM((G, 8, DIM), jnp.float32),
            pltpu.VMEM((G, 8, DIM), jnp.float32),
            pltpu.VMEM((16 * 17,), jnp.float32),
            pltpu.VMEM((B_PER_W,), jnp.float32),
            pltpu.SemaphoreType.DMA,
            pltpu.SemaphoreType.DMA,
        ],
    )
    return f(t_ids2, c_ids2, t_w3, c_w3)


# speedup vs baseline: 1.0321x; 1.0124x over previous
import jax
import jax.numpy as jnp
from jax import lax
from jax.experimental import pallas as pl
from jax.experimental.pallas import tpu as pltpu
from jax.experimental.pallas import tpu_sc as plsc

N_ROWS = 1000000
DIM = 64
BATCH = 16384
NW = 32
B_PER_W = BATCH // NW
G = 16
NCH = B_PER_W // G


def _sgns_body(t_ids_hbm, c_ids_hbm, t_w_hbm, c_w_hbm, out_hbm,
               idx_t_v, idx_c_v, q_t_v, q_c_v,
               bt0, bt1, bc0, bc1, part_v, out_v, sem0, sem1):
    wid = lax.axis_index("s") * 2 + lax.axis_index("c")

    pltpu.sync_copy(t_ids_hbm.at[wid], idx_t_v)
    pltpu.sync_copy(c_ids_hbm.at[wid], idx_c_v)
    for i in range(B_PER_W // 16):
        sl = pl.ds(i * 16, 16)
        q_t_v[sl] = lax.shift_right_logical(idx_t_v[sl], 1)
        q_c_v[sl] = lax.shift_right_logical(idx_c_v[sl], 1)

    bufs = ((bt0, bc0, sem0), (bt1, bc1, sem1))

    def fire(j, par):
        bt, bc, sem = bufs[par]
        pltpu.async_copy(t_w_hbm.at[q_t_v.at[pl.ds(j * G, G)]], bt, sem)
        pltpu.async_copy(c_w_hbm.at[q_c_v.at[pl.ds(j * G, G)]], bc, sem)

    def drain(par):
        bt, bc, sem = bufs[par]
        pltpu.make_async_copy(t_w_hbm.at[pl.ds(0, G)], bt, sem).wait()
        pltpu.make_async_copy(c_w_hbm.at[pl.ds(0, G)], bc, sem).wait()

    lane17 = lax.iota(jnp.int32, 16) * 17

    def compute(j, par):
        bt, bc, _ = bufs[par]
        sl = pl.ds(j * G, 16)
        ofs_t = lax.bitwise_and(idx_t_v[sl], 1) * 64
        ofs_c = lax.bitwise_and(idx_c_v[sl], 1) * 64
        for i in range(16):
            ot = ofs_t[i]
            oc = ofs_c[i]
            acc = (bt[i, pl.ds(ot, 16)] * bc[i, pl.ds(oc, 16)])
            for k in range(1, 4):
                acc = acc + (bt[i, pl.ds(ot + k * 16, 16)]
                             * bc[i, pl.ds(oc + k * 16, 16)])
            plsc.store_scatter(part_v, [lane17 + i], acc)
        s = part_v[pl.ds(0, 16)]
        for l in range(1, 16):
            s = s + part_v[pl.ds(l * 17, 16)]
        out_v[pl.ds(j * G, 16)] = s

    fire(0, 0)
    fire(1, 1)

    def body(g, _):
        j0 = g * 2
        drain(0)
        compute(j0, 0)

        @pl.when(g < NCH // 2 - 1)
        def _():
            fire(j0 + 2, 0)

        drain(1)
        compute(j0 + 1, 1)

        @pl.when(g < NCH // 2 - 1)
        def _():
            fire(j0 + 3, 1)

        return 0

    lax.fori_loop(0, NCH // 2, body, 0)

    pltpu.sync_copy(out_v, out_hbm.at[pl.ds(wid * B_PER_W, B_PER_W)])


@jax.jit
def kernel(t_ids, c_ids, t_in_w, c_out_w):
    t_ids2 = jnp.asarray(t_ids, jnp.int32).reshape(NW, B_PER_W)
    c_ids2 = jnp.asarray(c_ids, jnp.int32).reshape(NW, B_PER_W)
    t_w3 = lax.optimization_barrier(t_in_w.reshape(N_ROWS // 8, 8, DIM))
    c_w3 = lax.optimization_barrier(c_out_w.reshape(N_ROWS // 8, 8, DIM))
    t_w2 = t_w3.reshape(N_ROWS // 2, 2 * DIM)
    c_w2 = c_w3.reshape(N_ROWS // 2, 2 * DIM)

    mesh = plsc.VectorSubcoreMesh(core_axis_name="c", subcore_axis_name="s")
    f = pl.kernel(
        _sgns_body,
        out_type=jax.ShapeDtypeStruct((BATCH,), jnp.float32),
        mesh=mesh,
        compiler_params=pltpu.CompilerParams(needs_layout_passes=False),
        scratch_types=[
            pltpu.VMEM((B_PER_W,), jnp.int32),
            pltpu.VMEM((B_PER_W,), jnp.int32),
            pltpu.VMEM((B_PER_W,), jnp.int32),
            pltpu.VMEM((B_PER_W,), jnp.int32),
            pltpu.VMEM((G, 2 * DIM), jnp.float32),
            pltpu.VMEM((G, 2 * DIM), jnp.float32),
            pltpu.VMEM((G, 2 * DIM), jnp.float32),
            pltpu.VMEM((G, 2 * DIM), jnp.float32),
            pltpu.VMEM((16 * 17,), jnp.float32),
            pltpu.VMEM((B_PER_W,), jnp.float32),
            pltpu.SemaphoreType.DMA,
            pltpu.SemaphoreType.DMA,
        ],
    )
    return f(t_ids2, c_ids2, t_w2, c_w2)


# speedup vs baseline: 2.2940x; 2.2227x over previous
import jax
import jax.numpy as jnp
from jax import lax
from jax.experimental import pallas as pl
from jax.experimental.pallas import tpu as pltpu
from jax.experimental.pallas import tpu_sc as plsc

N_ROWS = 1000000
DIM = 64
BATCH = 16384
NW = 32
B_PER_W = BATCH // NW
C = 16
NCH = B_PER_W // C


def _sgns_body(t_ids_hbm, c_ids_hbm, t_w_hbm, c_w_hbm, out_hbm,
               idx_t_v, idx_c_v,
               bt0, bt1, bc0, bc1, part_v, out_v, sem0, sem1):
    wid = lax.axis_index("s") * 2 + lax.axis_index("c")

    pltpu.sync_copy(t_ids_hbm.at[wid], idx_t_v)
    pltpu.sync_copy(c_ids_hbm.at[wid], idx_c_v)

    bufs = ((bt0, bc0, sem0), (bt1, bc1, sem1))

    def fire(j, par):
        bt, bc, sem = bufs[par]
        for i16 in range(C // 16):
            sl = pl.ds(j * C + i16 * 16, 16)
            tqv = lax.shift_right_logical(idx_t_v[sl], 3)
            cqv = lax.shift_right_logical(idx_c_v[sl], 3)
            for i in range(16):
                e = i16 * 16 + i
                pltpu.async_copy(t_w_hbm.at[tqv[i]], bt.at[e], sem)
                pltpu.async_copy(c_w_hbm.at[cqv[i]], bc.at[e], sem)

    def drain(par):
        bt, bc, sem = bufs[par]
        pltpu.make_async_copy(t_w_hbm.at[pl.ds(0, C)], bt, sem).wait()
        pltpu.make_async_copy(c_w_hbm.at[pl.ds(0, C)], bc, sem).wait()

    lane17 = lax.iota(jnp.int32, 16) * 17

    def compute(j, par):
        bt, bc, _ = bufs[par]
        for i16 in range(C // 16):
            sl = pl.ds(j * C + i16 * 16, 16)
            sub_t = lax.bitwise_and(idx_t_v[sl], 7)
            sub_c = lax.bitwise_and(idx_c_v[sl], 7)
            for i in range(16):
                e = i16 * 16 + i
                st = sub_t[i]
                sc = sub_c[i]
                acc = (bt[e, st, pl.ds(0, 16)] * bc[e, sc, pl.ds(0, 16)])
                for k in range(1, 4):
                    acc = acc + (bt[e, st, pl.ds(k * 16, 16)]
                                 * bc[e, sc, pl.ds(k * 16, 16)])
                plsc.store_scatter(part_v, [lane17 + i], acc)
            s = part_v[pl.ds(0, 16)]
            for l in range(1, 16):
                s = s + part_v[pl.ds(l * 17, 16)]
            out_v[pl.ds(j * C + i16 * 16, 16)] = s

    fire(0, 0)
    fire(1, 1)

    def body(g, _):
        j0 = g * 2
        drain(0)
        compute(j0, 0)

        @pl.when(g < NCH // 2 - 1)
        def _():
            fire(j0 + 2, 0)

        drain(1)
        compute(j0 + 1, 1)

        @pl.when(g < NCH // 2 - 1)
        def _():
            fire(j0 + 3, 1)

        return 0

    lax.fori_loop(0, NCH // 2, body, 0)

    pltpu.sync_copy(out_v, out_hbm.at[pl.ds(wid * B_PER_W, B_PER_W)])


@jax.jit
def kernel(t_ids, c_ids, t_in_w, c_out_w):
    t_ids2 = jnp.asarray(t_ids, jnp.int32).reshape(NW, B_PER_W)
    c_ids2 = jnp.asarray(c_ids, jnp.int32).reshape(NW, B_PER_W)
    t_w3 = t_in_w.reshape(N_ROWS // 8, 8, DIM)
    c_w3 = c_out_w.reshape(N_ROWS // 8, 8, DIM)

    mesh = plsc.VectorSubcoreMesh(core_axis_name="c", subcore_axis_name="s")
    f = pl.kernel(
        _sgns_body,
        out_type=jax.ShapeDtypeStruct((BATCH,), jnp.float32),
        mesh=mesh,
        compiler_params=pltpu.CompilerParams(needs_layout_passes=False),
        scratch_types=[
            pltpu.VMEM((B_PER_W,), jnp.int32),
            pltpu.VMEM((B_PER_W,), jnp.int32),
            pltpu.VMEM((C, 8, DIM), jnp.float32),
            pltpu.VMEM((C, 8, DIM), jnp.float32),
            pltpu.VMEM((C, 8, DIM), jnp.float32),
            pltpu.VMEM((C, 8, DIM), jnp.float32),
            pltpu.VMEM((16 * 17,), jnp.float32),
            pltpu.VMEM((B_PER_W,), jnp.float32),
            pltpu.SemaphoreType.DMA,
            pltpu.SemaphoreType.DMA,
        ],
    )
    return f(t_ids2, c_ids2, t_w3, c_w3)


# speedup vs baseline: 2.3131x; 1.0083x over previous
import jax
import jax.numpy as jnp
from jax import lax
from jax.experimental import pallas as pl
from jax.experimental.pallas import tpu as pltpu
from jax.experimental.pallas import tpu_sc as plsc

N_ROWS = 1000000
DIM = 64
BATCH = 16384
NW = 32
B_PER_W = BATCH // NW
C = 16
NCH = B_PER_W // C


def _sgns_body(t_ids_hbm, c_ids_hbm, t_w_hbm, c_w_hbm, out_hbm,
               idx_t_v, idx_c_v,
               bt0, bt1, bt2, bc0, bc1, bc2, part_v, out_v,
               sem0, sem1, sem2):
    wid = lax.axis_index("s") * 2 + lax.axis_index("c")

    pltpu.sync_copy(t_ids_hbm.at[wid], idx_t_v)
    pltpu.sync_copy(c_ids_hbm.at[wid], idx_c_v)

    bufs = ((bt0, bc0, sem0), (bt1, bc1, sem1), (bt2, bc2, sem2))

    def fire(j, par):
        bt, bc, sem = bufs[par]
        for i16 in range(C // 16):
            sl = pl.ds(j * C + i16 * 16, 16)
            tqv = lax.shift_right_logical(idx_t_v[sl], 3)
            cqv = lax.shift_right_logical(idx_c_v[sl], 3)
            for i in range(16):
                e = i16 * 16 + i
                pltpu.async_copy(t_w_hbm.at[tqv[i]], bt.at[e], sem)
                pltpu.async_copy(c_w_hbm.at[cqv[i]], bc.at[e], sem)

    def drain(par):
        bt, bc, sem = bufs[par]
        pltpu.make_async_copy(t_w_hbm.at[pl.ds(0, C)], bt, sem).wait()
        pltpu.make_async_copy(c_w_hbm.at[pl.ds(0, C)], bc, sem).wait()

    lane17 = lax.iota(jnp.int32, 16) * 17

    def compute(j, par):
        bt, bc, _ = bufs[par]
        for i16 in range(C // 16):
            sl = pl.ds(j * C + i16 * 16, 16)
            sub_t = lax.bitwise_and(idx_t_v[sl], 7)
            sub_c = lax.bitwise_and(idx_c_v[sl], 7)
            for i in range(16):
                e = i16 * 16 + i
                st = sub_t[i]
                sc = sub_c[i]
                acc = (bt[e, st, pl.ds(0, 16)] * bc[e, sc, pl.ds(0, 16)])
                for k in range(1, 4):
                    acc = acc + (bt[e, st, pl.ds(k * 16, 16)]
                                 * bc[e, sc, pl.ds(k * 16, 16)])
                plsc.store_scatter(part_v, [lane17 + i], acc)
            s = part_v[pl.ds(0, 16)]
            for l in range(1, 16):
                s = s + part_v[pl.ds(l * 17, 16)]
            out_v[pl.ds(j * C + i16 * 16, 16)] = s

    fire(0, 0)
    fire(1, 1)
    fire(2, 2)

    def body(g, _):
        j0 = g * 3
        for p in range(3):
            drain(p)
            compute(j0 + p, p)

            @pl.when(j0 + p + 3 < NCH)
            def _():
                fire(j0 + p + 3, p)

        return 0

    lax.fori_loop(0, NCH // 3, body, 0)
    drain(0)
    compute(NCH - 2, 0)
    drain(1)
    compute(NCH - 1, 1)

    pltpu.sync_copy(out_v, out_hbm.at[pl.ds(wid * B_PER_W, B_PER_W)])


@jax.jit
def kernel(t_ids, c_ids, t_in_w, c_out_w):
    t_ids2 = jnp.asarray(t_ids, jnp.int32).reshape(NW, B_PER_W)
    c_ids2 = jnp.asarray(c_ids, jnp.int32).reshape(NW, B_PER_W)
    t_w3 = t_in_w.reshape(N_ROWS // 8, 8, DIM)
    c_w3 = c_out_w.reshape(N_ROWS // 8, 8, DIM)

    mesh = plsc.VectorSubcoreMesh(core_axis_name="c", subcore_axis_name="s")
    f = pl.kernel(
        _sgns_body,
        out_type=jax.ShapeDtypeStruct((BATCH,), jnp.float32),
        mesh=mesh,
        compiler_params=pltpu.CompilerParams(needs_layout_passes=False),
        scratch_types=[
            pltpu.VMEM((B_PER_W,), jnp.int32),
            pltpu.VMEM((B_PER_W,), jnp.int32),
            pltpu.VMEM((C, 8, DIM), jnp.float32),
            pltpu.VMEM((C, 8, DIM), jnp.float32),
            pltpu.VMEM((C, 8, DIM), jnp.float32),
            pltpu.VMEM((C, 8, DIM), jnp.float32),
            pltpu.VMEM((C, 8, DIM), jnp.float32),
            pltpu.VMEM((C, 8, DIM), jnp.float32),
            pltpu.VMEM((16 * 17,), jnp.float32),
            pltpu.VMEM((B_PER_W,), jnp.float32),
            pltpu.SemaphoreType.DMA,
            pltpu.SemaphoreType.DMA,
            pltpu.SemaphoreType.DMA,
        ],
    )
    return f(t_ids2, c_ids2, t_w3, c_w3)
